# Initial kernel scaffold; baseline (speedup 1.0000x reference)
import jax
import jax.numpy as jnp
from jax.experimental import pallas as pl


def kernel(portfolio_value):
    raise NotImplementedError("write your pallas kernel here")


# speedup vs baseline: 22.1236x; 22.1236x over previous
import functools

import jax
import jax.numpy as jnp
import numpy as np
from jax import lax
from jax.experimental import pallas as pl
from jax.experimental.pallas import tpu as pltpu
from jax.experimental.pallas import tpu_sc as plsc

NS = 16
LANES = 16
MININT = np.int32(-2147483648)
INF_BITS = 0x7F800000


def _to_key(x):
  return jnp.where(x < 0, ~x, x ^ MININT)


def _median_sc(n_pad, rank):
  chunk = n_pad // NS
  nv = chunk // LANES

  mesh = plsc.VectorSubcoreMesh(core_axis_name="c", subcore_axis_name="s",
                                num_cores=1)

  @functools.partial(
      pl.kernel,
      out_type=jax.ShapeDtypeStruct((LANES,), jnp.float32),
      mesh=mesh,
      compiler_params=pltpu.CompilerParams(needs_layout_passes=False),
      scratch_types=[
          pltpu.VMEM((chunk,), jnp.int32),
          pltpu.VMEM((LANES * 256,), jnp.int32),
          pltpu.VMEM((256,), jnp.int32),
          pltpu.VMEM((NS, 256), jnp.int32),
          pltpu.VMEM_SHARED((NS, 256), jnp.int32),
          pltpu.VMEM((LANES,), jnp.float32),
      ],
  )
  def body(x_hbm, out_hbm, xb, hist, cnt, gbuf, shared, obuf):
    sid = lax.axis_index("s")
    base = sid * chunk
    pltpu.sync_copy(x_hbm.at[pl.ds(base, chunk)], xb)

    lane_base = lax.iota(jnp.int32, LANES) * 256
    ones = jnp.ones((LANES,), jnp.int32)
    zeros = jnp.zeros((LANES,), jnp.int32)

    prefix = np.int32(0)
    rk = np.int32(rank)

    for r in range(4):
      shift = 24 - 8 * r

      @plsc.parallel_loop(0, 256, 1, unroll=8)
      def _(j):
        hist[pl.ds(j * LANES, LANES)] = zeros

      if r == 0:
        @plsc.parallel_loop(0, nv, 1, unroll=8)
        def _(i):
          key = _to_key(xb[pl.ds(i * LANES, LANES)])
          b = lax.shift_right_logical(key, shift) & 255
          plsc.addupdate_scatter(hist, [b + lane_base], ones)
      else:
        pfx = prefix

        @plsc.parallel_loop(0, nv, 1, unroll=8)
        def _(i):
          key = _to_key(xb[pl.ds(i * LANES, LANES)])
          b = lax.shift_right_logical(key, shift) & 255
          m = lax.shift_right_logical(key, shift + 8) == pfx
          plsc.addupdate_scatter(hist, [b + lane_base], ones, mask=m)

      @plsc.parallel_loop(0, 16, 1)
      def _(j):
        acc = hist[pl.ds(j * LANES, LANES)]
        for c in range(1, LANES):
          acc = acc + hist[pl.ds(c * 256 + j * LANES, LANES)]
        cnt[pl.ds(j * LANES, LANES)] = acc

      pltpu.sync_copy(cnt, shared.at[sid])
      plsc.subcore_barrier()
      pltpu.sync_copy(shared, gbuf)
      plsc.subcore_barrier()

      carry = np.int32(0)
      bstar = np.int32(0)
      below = np.int32(0)
      for j in range(16):
        g = gbuf[0, pl.ds(j * LANES, LANES)]
        for t in range(1, NS):
          g = g + gbuf[t, pl.ds(j * LANES, LANES)]
        s = jnp.cumsum(g) + carry
        m = s <= rk
        bstar = bstar + jnp.sum(m.astype(jnp.int32))
        below = jnp.maximum(below, jnp.max(jnp.where(m, s, 0)))
        carry = jnp.max(s)

      rk = rk - below
      prefix = lax.shift_left(prefix, 8) + bstar

    xbits = jnp.where(prefix < 0, prefix ^ MININT, ~prefix)
    obuf[...] = plsc.bitcast(xbits + jnp.zeros((LANES,), jnp.int32),
                             jnp.float32)

    @pl.when(sid == 0)
    def _():
      pltpu.sync_copy(obuf, out_hbm)

  return body


def kernel(portfolio_value):
  flat = portfolio_value.reshape(-1)
  n = flat.shape[0]
  rank = (n - 1) // 2
  gran = NS * LANES
  n_pad = ((n + gran - 1) // gran) * gran
  xi = lax.bitcast_convert_type(flat, jnp.int32)
  if n_pad != n:
    xi = jnp.pad(xi, (0, n_pad - n), constant_values=INF_BITS)
  out = _median_sc(n_pad, rank)(xi)
  return out[0]
